# baseline (speedup 1.0000x reference)
import functools

import jax
import jax.numpy as jnp
from jax import lax
from jax.experimental import pallas as pl
from jax.experimental.pallas import tpu as pltpu
from jax.experimental.pallas import tpu_sc as plsc

N, E, F, H, G = 10000, 320000, 128, 64, 128

NUM_CORES = 2
NUM_SUBCORES = 16
NUM_WORKERS = NUM_CORES * NUM_SUBCORES
EDGES_PER_TILE = E // NUM_WORKERS
CHUNK = 80
CHUNKS = EDGES_PER_TILE // CHUNK
NBUF = 5
GROUPS = CHUNKS // NBUF
ROWS_BIG = 632
ROWS_LAST = N - 15 * ROWS_BIG

_HIGH = jax.lax.Precision.HIGHEST


def _agg_body(y_hbm, srcs_hbm, dsts_hbm, zeros_hbm, out_hbm,
              src_v, dst_v, rows_v, acc_sh, gsem, ssem):
    c = lax.axis_index("c")
    s = lax.axis_index("s")
    base = s * ROWS_BIG
    pltpu.sync_copy(srcs_hbm.at[c].at[s], src_v)
    pltpu.sync_copy(dsts_hbm.at[c].at[s], dst_v)

    @pl.when(s < 15)
    def _():
        pltpu.sync_copy(zeros_hbm, acc_sh.at[pl.ds(base, ROWS_BIG)])

    @pl.when(s == 15)
    def _():
        pltpu.sync_copy(zeros_hbm.at[pl.ds(0, ROWS_LAST)],
                        acc_sh.at[pl.ds(15 * ROWS_BIG, ROWS_LAST)])

    plsc.subcore_barrier()

    def step(g, carry):
        gds = []
        for b in range(NBUF):
            j = g * NBUF + b
            gds.append(pltpu.async_copy(y_hbm.at[src_v.at[j]],
                                        rows_v.at[b], gsem.at[b]))
        sds = []
        for b in range(NBUF):
            j = g * NBUF + b
            gds[b].wait()
            sds.append(pltpu.async_copy(rows_v.at[b], acc_sh.at[dst_v.at[j]],
                                        ssem.at[b], add=True))
        for b in range(NBUF):
            sds[b].wait()
        return carry

    lax.fori_loop(0, GROUPS, step, 0)
    plsc.subcore_barrier()

    @pl.when(s < 15)
    def _():
        pltpu.sync_copy(acc_sh.at[pl.ds(base, ROWS_BIG)],
                        out_hbm.at[c].at[pl.ds(base, ROWS_BIG)])

    @pl.when(s == 15)
    def _():
        pltpu.sync_copy(acc_sh.at[pl.ds(15 * ROWS_BIG, ROWS_LAST)],
                        out_hbm.at[c].at[pl.ds(15 * ROWS_BIG, ROWS_LAST)])


_agg = pl.kernel(
    _agg_body,
    out_type=jax.ShapeDtypeStruct((NUM_CORES, N, H), jnp.float32),
    mesh=plsc.VectorSubcoreMesh(core_axis_name="c", subcore_axis_name="s"),
    scratch_types=[
        pltpu.VMEM((CHUNKS, CHUNK), jnp.int32),
        pltpu.VMEM((CHUNKS, CHUNK), jnp.int32),
        pltpu.VMEM((NBUF, CHUNK, H), jnp.float32),
        pltpu.VMEM_SHARED((N, H), jnp.float32),
        pltpu.SemaphoreType.DMA((NBUF,)),
        pltpu.SemaphoreType.DMA((NBUF,)),
    ],
    compiler_params=pltpu.CompilerParams(use_tc_tiling_on_sc=False),
)


_BLK = 2000


def _layer1_body(x_ref, plo0, plo1, phi0, phi1, wa_ref, ba_ref, wb_ref,
                 bb_ref, o_ref):
    agg = jnp.concatenate([plo0[...] + plo1[...], phi0[...] + phi1[...]],
                          axis=1)
    t = x_ref[...] + agg
    t = jnp.dot(t, wa_ref[...], preferred_element_type=jnp.float32) + ba_ref[...]
    t = jnp.maximum(t, 0.0)
    t = jnp.dot(t, wb_ref[...], preferred_element_type=jnp.float32) + bb_ref[...]
    o_ref[...] = jnp.maximum(t, 0.0)


_layer1 = pl.pallas_call(
    _layer1_body,
    grid=(N // _BLK,),
    in_specs=[
        pl.BlockSpec((_BLK, F), lambda i: (i, 0)),
        pl.BlockSpec((_BLK, H), lambda i: (i, 0)),
        pl.BlockSpec((_BLK, H), lambda i: (i, 0)),
        pl.BlockSpec((_BLK, H), lambda i: (i, 0)),
        pl.BlockSpec((_BLK, H), lambda i: (i, 0)),
        pl.BlockSpec((F, H), lambda i: (0, 0)),
        pl.BlockSpec((1, H), lambda i: (0, 0)),
        pl.BlockSpec((H, H), lambda i: (0, 0)),
        pl.BlockSpec((1, H), lambda i: (0, 0)),
    ],
    out_specs=pl.BlockSpec((_BLK, H), lambda i: (i, 0)),
    out_shape=jax.ShapeDtypeStruct((N, H), jnp.float32),
)


def _layer_body(h_ref, p0_ref, p1_ref, wa_ref, ba_ref, wb_ref, bb_ref,
                o_ref, *, final_relu):
    t = h_ref[...] + p0_ref[...] + p1_ref[...]
    t = jnp.dot(t, wa_ref[...], preferred_element_type=jnp.float32) + ba_ref[...]
    t = jnp.maximum(t, 0.0)
    t = jnp.dot(t, wb_ref[...], preferred_element_type=jnp.float32) + bb_ref[...]
    o_ref[...] = jnp.maximum(t, 0.0) if final_relu else t


def _make_layer(final_relu):
    return pl.pallas_call(
        functools.partial(_layer_body, final_relu=final_relu),
        grid=(N // _BLK,),
        in_specs=[
            pl.BlockSpec((_BLK, H), lambda i: (i, 0)),
            pl.BlockSpec((_BLK, H), lambda i: (i, 0)),
            pl.BlockSpec((_BLK, H), lambda i: (i, 0)),
            pl.BlockSpec((H, H), lambda i: (0, 0)),
            pl.BlockSpec((1, H), lambda i: (0, 0)),
            pl.BlockSpec((H, H), lambda i: (0, 0)),
            pl.BlockSpec((1, H), lambda i: (0, 0)),
        ],
        out_specs=pl.BlockSpec((_BLK, H), lambda i: (i, 0)),
        out_shape=jax.ShapeDtypeStruct((N, H), jnp.float32),
    )


_layer2 = _make_layer(True)
_layer3 = _make_layer(False)


def _pool_body(h_ref, batch_ref, wf_ref, bf_ref, o_ref):
    gid = lax.broadcasted_iota(jnp.int32, (G, N), 0)
    m = (gid == batch_ref[...]).astype(jnp.float32)
    sums = jnp.dot(m, h_ref[...], preferred_element_type=jnp.float32,
                   precision=_HIGH)
    counts = jnp.sum(m, axis=1, keepdims=True)
    pooled = sums / jnp.maximum(counts, 1.0)
    o_ref[...] = jnp.dot(pooled, wf_ref[...],
                         preferred_element_type=jnp.float32) + bf_ref[...]


_pool = pl.pallas_call(
    _pool_body,
    in_specs=[
        pl.BlockSpec((N, H), lambda: (0, 0)),
        pl.BlockSpec((1, N), lambda: (0, 0)),
        pl.BlockSpec((H, 1), lambda: (0, 0)),
        pl.BlockSpec((1, 1), lambda: (0, 0)),
    ],
    out_specs=pl.BlockSpec((G, 1), lambda: (0, 0)),
    out_shape=jax.ShapeDtypeStruct((G, 1), jnp.float32),
)


def kernel(x, edge_index, batch, W1a, b1a, W1b, b1b, W2a, b2a, W2b, b2b,
           W3a, b3a, W3b, b3b, Wf, bf):
    shp = (NUM_CORES, NUM_SUBCORES, CHUNKS, CHUNK)
    src = edge_index[0]
    src_r = src.reshape(shp)
    dst_r = edge_index[1].reshape(shp)
    zeros_h = jnp.zeros((ROWS_BIG, H), jnp.float32)
    batch2d = batch.reshape(1, N)

    x2 = x.reshape(2 * N, H)
    p_lo = _agg(x2, (src * 2).reshape(shp), dst_r, zeros_h)
    p_hi = _agg(x2, (src * 2 + 1).reshape(shp), dst_r, zeros_h)
    h1 = _layer1(x, p_lo[0], p_lo[1], p_hi[0], p_hi[1],
                 W1a, b1a.reshape(1, H), W1b, b1b.reshape(1, H))

    p = _agg(h1, src_r, dst_r, zeros_h)
    h2 = _layer2(h1, p[0], p[1], W2a, b2a.reshape(1, H), W2b, b2b.reshape(1, H))

    p = _agg(h2, src_r, dst_r, zeros_h)
    h3 = _layer3(h2, p[0], p[1], W3a, b3a.reshape(1, H), W3b, b3b.reshape(1, H))

    return _pool(h3, batch2d, Wf, bf.reshape(1, 1))


# speedup vs baseline: 1.0272x; 1.0272x over previous
import functools

import jax
import jax.numpy as jnp
from jax import lax
from jax.experimental import pallas as pl
from jax.experimental.pallas import tpu as pltpu
from jax.experimental.pallas import tpu_sc as plsc

N, E, F, H, G = 10000, 320000, 128, 64, 128

NUM_CORES = 2
NUM_SUBCORES = 16
NUM_WORKERS = NUM_CORES * NUM_SUBCORES
EDGES_PER_TILE = E // NUM_WORKERS
CHUNK = 80
CHUNKS = EDGES_PER_TILE // CHUNK
NBUF = 5
GROUPS = CHUNKS // NBUF
ROWS_BIG = 632
ROWS_LAST = N - 15 * ROWS_BIG

_HIGH = jax.lax.Precision.HIGHEST


def _agg_body(y_hbm, srcs_hbm, dsts_hbm, zeros_hbm, out_hbm,
              src_v, dst_v, rows_v, acc_sh, gsem, ssem, *, groups):
    c = lax.axis_index("c")
    s = lax.axis_index("s")
    base = s * ROWS_BIG
    pltpu.sync_copy(srcs_hbm.at[c].at[s], src_v)
    pltpu.sync_copy(dsts_hbm.at[c].at[s], dst_v)

    @pl.when(s < 15)
    def _():
        pltpu.sync_copy(zeros_hbm, acc_sh.at[pl.ds(base, ROWS_BIG)])

    @pl.when(s == 15)
    def _():
        pltpu.sync_copy(zeros_hbm.at[pl.ds(0, ROWS_LAST)],
                        acc_sh.at[pl.ds(15 * ROWS_BIG, ROWS_LAST)])

    plsc.subcore_barrier()

    def step(g, carry):
        gds = []
        for b in range(NBUF):
            j = g * NBUF + b
            gds.append(pltpu.async_copy(y_hbm.at[src_v.at[j]],
                                        rows_v.at[b], gsem.at[b]))
        sds = []
        for b in range(NBUF):
            j = g * NBUF + b
            gds[b].wait()
            sds.append(pltpu.async_copy(rows_v.at[b], acc_sh.at[dst_v.at[j]],
                                        ssem.at[b], add=True))
        for b in range(NBUF):
            sds[b].wait()
        return carry

    lax.fori_loop(0, groups, step, 0)
    plsc.subcore_barrier()

    @pl.when(s < 15)
    def _():
        pltpu.sync_copy(acc_sh.at[pl.ds(base, ROWS_BIG)],
                        out_hbm.at[c].at[pl.ds(base, ROWS_BIG)])

    @pl.when(s == 15)
    def _():
        pltpu.sync_copy(acc_sh.at[pl.ds(15 * ROWS_BIG, ROWS_LAST)],
                        out_hbm.at[c].at[pl.ds(15 * ROWS_BIG, ROWS_LAST)])


def _make_agg(chunks):
    return pl.kernel(
        functools.partial(_agg_body, groups=chunks // NBUF),
        out_type=jax.ShapeDtypeStruct((NUM_CORES, N, H), jnp.float32),
        mesh=plsc.VectorSubcoreMesh(core_axis_name="c", subcore_axis_name="s"),
        scratch_types=[
            pltpu.VMEM((chunks, CHUNK), jnp.int32),
            pltpu.VMEM((chunks, CHUNK), jnp.int32),
            pltpu.VMEM((NBUF, CHUNK, H), jnp.float32),
            pltpu.VMEM_SHARED((N, H), jnp.float32),
            pltpu.SemaphoreType.DMA((NBUF,)),
            pltpu.SemaphoreType.DMA((NBUF,)),
        ],
        compiler_params=pltpu.CompilerParams(use_tc_tiling_on_sc=False),
    )


_agg = _make_agg(CHUNKS)
_agg_half = _make_agg(2 * CHUNKS)


_BLK = 2000


def _layer1_body(x_ref, plo, phi, wa_ref, ba_ref, wb_ref, bb_ref, o_ref):
    agg = jnp.concatenate([plo[...], phi[...]], axis=1)
    t = x_ref[...] + agg
    t = jnp.dot(t, wa_ref[...], preferred_element_type=jnp.float32) + ba_ref[...]
    t = jnp.maximum(t, 0.0)
    t = jnp.dot(t, wb_ref[...], preferred_element_type=jnp.float32) + bb_ref[...]
    o_ref[...] = jnp.maximum(t, 0.0)


_layer1 = pl.pallas_call(
    _layer1_body,
    grid=(N // _BLK,),
    in_specs=[
        pl.BlockSpec((_BLK, F), lambda i: (i, 0)),
        pl.BlockSpec((_BLK, H), lambda i: (i, 0)),
        pl.BlockSpec((_BLK, H), lambda i: (i, 0)),
        pl.BlockSpec((F, H), lambda i: (0, 0)),
        pl.BlockSpec((1, H), lambda i: (0, 0)),
        pl.BlockSpec((H, H), lambda i: (0, 0)),
        pl.BlockSpec((1, H), lambda i: (0, 0)),
    ],
    out_specs=pl.BlockSpec((_BLK, H), lambda i: (i, 0)),
    out_shape=jax.ShapeDtypeStruct((N, H), jnp.float32),
)


def _layer_body(h_ref, p0_ref, p1_ref, wa_ref, ba_ref, wb_ref, bb_ref,
                o_ref, *, final_relu):
    t = h_ref[...] + p0_ref[...] + p1_ref[...]
    t = jnp.dot(t, wa_ref[...], preferred_element_type=jnp.float32) + ba_ref[...]
    t = jnp.maximum(t, 0.0)
    t = jnp.dot(t, wb_ref[...], preferred_element_type=jnp.float32) + bb_ref[...]
    o_ref[...] = jnp.maximum(t, 0.0) if final_relu else t


def _make_layer(final_relu):
    return pl.pallas_call(
        functools.partial(_layer_body, final_relu=final_relu),
        grid=(N // _BLK,),
        in_specs=[
            pl.BlockSpec((_BLK, H), lambda i: (i, 0)),
            pl.BlockSpec((_BLK, H), lambda i: (i, 0)),
            pl.BlockSpec((_BLK, H), lambda i: (i, 0)),
            pl.BlockSpec((H, H), lambda i: (0, 0)),
            pl.BlockSpec((1, H), lambda i: (0, 0)),
            pl.BlockSpec((H, H), lambda i: (0, 0)),
            pl.BlockSpec((1, H), lambda i: (0, 0)),
        ],
        out_specs=pl.BlockSpec((_BLK, H), lambda i: (i, 0)),
        out_shape=jax.ShapeDtypeStruct((N, H), jnp.float32),
    )


_layer2 = _make_layer(True)
_layer3 = _make_layer(False)


def _pool_body(h_ref, batch_ref, wf_ref, bf_ref, o_ref):
    gid = lax.broadcasted_iota(jnp.int32, (G, N), 0)
    m = (gid == batch_ref[...]).astype(jnp.float32)
    sums = jnp.dot(m, h_ref[...], preferred_element_type=jnp.float32,
                   precision=_HIGH)
    counts = jnp.sum(m, axis=1, keepdims=True)
    pooled = sums / jnp.maximum(counts, 1.0)
    o_ref[...] = jnp.dot(pooled, wf_ref[...],
                         preferred_element_type=jnp.float32) + bf_ref[...]


_pool = pl.pallas_call(
    _pool_body,
    in_specs=[
        pl.BlockSpec((N, H), lambda: (0, 0)),
        pl.BlockSpec((1, N), lambda: (0, 0)),
        pl.BlockSpec((H, 1), lambda: (0, 0)),
        pl.BlockSpec((1, 1), lambda: (0, 0)),
    ],
    out_specs=pl.BlockSpec((G, 1), lambda: (0, 0)),
    out_shape=jax.ShapeDtypeStruct((G, 1), jnp.float32),
)


def kernel(x, edge_index, batch, W1a, b1a, W1b, b1b, W2a, b2a, W2b, b2b,
           W3a, b3a, W3b, b3b, Wf, bf):
    shp = (NUM_CORES, NUM_SUBCORES, CHUNKS, CHUNK)
    src = edge_index[0]
    src_r = src.reshape(shp)
    dst_r = edge_index[1].reshape(shp)
    zeros_h = jnp.zeros((ROWS_BIG, H), jnp.float32)
    batch2d = batch.reshape(1, N)

    x2 = x.reshape(2 * N, H)
    half_shp = (NUM_SUBCORES, 2 * CHUNKS, CHUNK)
    src2 = src * 2
    src_both = jnp.stack([src2.reshape(half_shp),
                          (src2 + 1).reshape(half_shp)])
    dst_m = edge_index[1].reshape(half_shp)
    dst_both = jnp.stack([dst_m, dst_m])
    p = _agg_half(x2, src_both, dst_both, zeros_h)
    h1 = _layer1(x, p[0], p[1],
                 W1a, b1a.reshape(1, H), W1b, b1b.reshape(1, H))

    p = _agg(h1, src_r, dst_r, zeros_h)
    h2 = _layer2(h1, p[0], p[1], W2a, b2a.reshape(1, H), W2b, b2b.reshape(1, H))

    p = _agg(h2, src_r, dst_r, zeros_h)
    h3 = _layer3(h2, p[0], p[1], W3a, b3a.reshape(1, H), W3b, b3b.reshape(1, H))

    return _pool(h3, batch2d, Wf, bf.reshape(1, 1))


# speedup vs baseline: 1.1959x; 1.1643x over previous
import functools

import jax
import jax.numpy as jnp
from jax import lax
from jax.experimental import pallas as pl
from jax.experimental.pallas import tpu as pltpu
from jax.experimental.pallas import tpu_sc as plsc

N, E, F, H, G = 10000, 320000, 128, 64, 128

NUM_CORES = 2
NUM_SUBCORES = 16
NUM_WORKERS = NUM_CORES * NUM_SUBCORES
EDGES_PER_TILE = E // NUM_WORKERS
CHUNK = 80
CHUNKS = EDGES_PER_TILE // CHUNK
NBUF = 5
GROUPS = CHUNKS // NBUF
ROWS_BIG = 632
ROWS_LAST = N - 15 * ROWS_BIG

_HIGH = jax.lax.Precision.HIGHEST


def _agg_body(y_hbm, srcs_hbm, dsts_hbm, zeros_hbm, out_hbm,
              src_v, dst_v, rows_v, acc_sh, gsem, ssem, *, groups):
    c = lax.axis_index("c")
    s = lax.axis_index("s")
    base = s * ROWS_BIG
    pltpu.sync_copy(srcs_hbm.at[c].at[s], src_v)
    pltpu.sync_copy(dsts_hbm.at[c].at[s], dst_v)

    @pl.when(s < 15)
    def _():
        pltpu.sync_copy(zeros_hbm, acc_sh.at[pl.ds(base, ROWS_BIG)])

    @pl.when(s == 15)
    def _():
        pltpu.sync_copy(zeros_hbm.at[pl.ds(0, ROWS_LAST)],
                        acc_sh.at[pl.ds(15 * ROWS_BIG, ROWS_LAST)])

    plsc.subcore_barrier()

    def step(g, carry):
        gds = []
        for b in range(NBUF):
            j = g * NBUF + b

            @pl.when(g > 0)
            def _(b=b, j=j):
                pltpu.make_async_copy(rows_v.at[b],
                                      acc_sh.at[dst_v.at[j - NBUF]],
                                      ssem.at[b]).wait()

            gds.append(pltpu.async_copy(y_hbm.at[src_v.at[j]],
                                        rows_v.at[b], gsem.at[b]))
        for b in range(NBUF):
            j = g * NBUF + b
            gds[b].wait()
            pltpu.async_copy(rows_v.at[b], acc_sh.at[dst_v.at[j]],
                             ssem.at[b], add=True)
        return carry

    lax.fori_loop(0, groups, step, 0)
    for b in range(NBUF):
        j = (groups - 1) * NBUF + b
        pltpu.make_async_copy(rows_v.at[b], acc_sh.at[dst_v.at[j]],
                              ssem.at[b]).wait()
    plsc.subcore_barrier()

    @pl.when(s < 15)
    def _():
        pltpu.sync_copy(acc_sh.at[pl.ds(base, ROWS_BIG)],
                        out_hbm.at[c].at[pl.ds(base, ROWS_BIG)])

    @pl.when(s == 15)
    def _():
        pltpu.sync_copy(acc_sh.at[pl.ds(15 * ROWS_BIG, ROWS_LAST)],
                        out_hbm.at[c].at[pl.ds(15 * ROWS_BIG, ROWS_LAST)])


def _make_agg(chunks):
    return pl.kernel(
        functools.partial(_agg_body, groups=chunks // NBUF),
        out_type=jax.ShapeDtypeStruct((NUM_CORES, N, H), jnp.float32),
        mesh=plsc.VectorSubcoreMesh(core_axis_name="c", subcore_axis_name="s"),
        scratch_types=[
            pltpu.VMEM((chunks, CHUNK), jnp.int32),
            pltpu.VMEM((chunks, CHUNK), jnp.int32),
            pltpu.VMEM((NBUF, CHUNK, H), jnp.float32),
            pltpu.VMEM_SHARED((N, H), jnp.float32),
            pltpu.SemaphoreType.DMA((NBUF,)),
            pltpu.SemaphoreType.DMA((NBUF,)),
        ],
        compiler_params=pltpu.CompilerParams(use_tc_tiling_on_sc=False),
    )


_agg = _make_agg(CHUNKS)
_agg_half = _make_agg(2 * CHUNKS)


_BLK = 2000


def _layer1_body(x_ref, plo, phi, wa_ref, ba_ref, wb_ref, bb_ref, o_ref):
    agg = jnp.concatenate([plo[...], phi[...]], axis=1)
    t = x_ref[...] + agg
    t = jnp.dot(t, wa_ref[...], preferred_element_type=jnp.float32) + ba_ref[...]
    t = jnp.maximum(t, 0.0)
    t = jnp.dot(t, wb_ref[...], preferred_element_type=jnp.float32) + bb_ref[...]
    o_ref[...] = jnp.maximum(t, 0.0)


_layer1 = pl.pallas_call(
    _layer1_body,
    grid=(N // _BLK,),
    in_specs=[
        pl.BlockSpec((_BLK, F), lambda i: (i, 0)),
        pl.BlockSpec((_BLK, H), lambda i: (i, 0)),
        pl.BlockSpec((_BLK, H), lambda i: (i, 0)),
        pl.BlockSpec((F, H), lambda i: (0, 0)),
        pl.BlockSpec((1, H), lambda i: (0, 0)),
        pl.BlockSpec((H, H), lambda i: (0, 0)),
        pl.BlockSpec((1, H), lambda i: (0, 0)),
    ],
    out_specs=pl.BlockSpec((_BLK, H), lambda i: (i, 0)),
    out_shape=jax.ShapeDtypeStruct((N, H), jnp.float32),
)


def _layer_body(h_ref, p0_ref, p1_ref, wa_ref, ba_ref, wb_ref, bb_ref,
                o_ref, *, final_relu):
    t = h_ref[...] + p0_ref[...] + p1_ref[...]
    t = jnp.dot(t, wa_ref[...], preferred_element_type=jnp.float32) + ba_ref[...]
    t = jnp.maximum(t, 0.0)
    t = jnp.dot(t, wb_ref[...], preferred_element_type=jnp.float32) + bb_ref[...]
    o_ref[...] = jnp.maximum(t, 0.0) if final_relu else t


def _make_layer(final_relu):
    return pl.pallas_call(
        functools.partial(_layer_body, final_relu=final_relu),
        grid=(N // _BLK,),
        in_specs=[
            pl.BlockSpec((_BLK, H), lambda i: (i, 0)),
            pl.BlockSpec((_BLK, H), lambda i: (i, 0)),
            pl.BlockSpec((_BLK, H), lambda i: (i, 0)),
            pl.BlockSpec((H, H), lambda i: (0, 0)),
            pl.BlockSpec((1, H), lambda i: (0, 0)),
            pl.BlockSpec((H, H), lambda i: (0, 0)),
            pl.BlockSpec((1, H), lambda i: (0, 0)),
        ],
        out_specs=pl.BlockSpec((_BLK, H), lambda i: (i, 0)),
        out_shape=jax.ShapeDtypeStruct((N, H), jnp.float32),
    )


_layer2 = _make_layer(True)
_layer3 = _make_layer(False)


def _pool_body(h_ref, batch_ref, wf_ref, bf_ref, o_ref):
    gid = lax.broadcasted_iota(jnp.int32, (G, N), 0)
    m = (gid == batch_ref[...]).astype(jnp.float32)
    sums = jnp.dot(m, h_ref[...], preferred_element_type=jnp.float32,
                   precision=_HIGH)
    counts = jnp.sum(m, axis=1, keepdims=True)
    pooled = sums / jnp.maximum(counts, 1.0)
    o_ref[...] = jnp.dot(pooled, wf_ref[...],
                         preferred_element_type=jnp.float32) + bf_ref[...]


_pool = pl.pallas_call(
    _pool_body,
    in_specs=[
        pl.BlockSpec((N, H), lambda: (0, 0)),
        pl.BlockSpec((1, N), lambda: (0, 0)),
        pl.BlockSpec((H, 1), lambda: (0, 0)),
        pl.BlockSpec((1, 1), lambda: (0, 0)),
    ],
    out_specs=pl.BlockSpec((G, 1), lambda: (0, 0)),
    out_shape=jax.ShapeDtypeStruct((G, 1), jnp.float32),
)


def kernel(x, edge_index, batch, W1a, b1a, W1b, b1b, W2a, b2a, W2b, b2b,
           W3a, b3a, W3b, b3b, Wf, bf):
    shp = (NUM_CORES, NUM_SUBCORES, CHUNKS, CHUNK)
    src = edge_index[0]
    src_r = src.reshape(shp)
    dst_r = edge_index[1].reshape(shp)
    zeros_h = jnp.zeros((ROWS_BIG, H), jnp.float32)
    batch2d = batch.reshape(1, N)

    x2 = x.reshape(2 * N, H)
    half_shp = (NUM_SUBCORES, 2 * CHUNKS, CHUNK)
    src2 = src * 2
    src_both = jnp.stack([src2.reshape(half_shp),
                          (src2 + 1).reshape(half_shp)])
    dst_m = edge_index[1].reshape(half_shp)
    dst_both = jnp.stack([dst_m, dst_m])
    p = _agg_half(x2, src_both, dst_both, zeros_h)
    h1 = _layer1(x, p[0], p[1],
                 W1a, b1a.reshape(1, H), W1b, b1b.reshape(1, H))

    p = _agg(h1, src_r, dst_r, zeros_h)
    h2 = _layer2(h1, p[0], p[1], W2a, b2a.reshape(1, H), W2b, b2b.reshape(1, H))

    p = _agg(h2, src_r, dst_r, zeros_h)
    h3 = _layer3(h2, p[0], p[1], W3a, b3a.reshape(1, H), W3b, b3b.reshape(1, H))

    return _pool(h3, batch2d, Wf, bf.reshape(1, 1))


# speedup vs baseline: 1.2970x; 1.0845x over previous
import functools

import jax
import jax.numpy as jnp
from jax import lax
from jax.experimental import pallas as pl
from jax.experimental.pallas import tpu as pltpu
from jax.experimental.pallas import tpu_sc as plsc

N, E, F, H, G = 10000, 320000, 128, 64, 128

NUM_CORES = 2
NUM_SUBCORES = 16
NUM_WORKERS = NUM_CORES * NUM_SUBCORES
EDGES_PER_TILE = E // NUM_WORKERS
CHUNK = 80
CHUNKS = EDGES_PER_TILE // CHUNK
NBUF = 5
GROUPS = CHUNKS // NBUF
ROWS_BIG = 632
ROWS_LAST = N - 15 * ROWS_BIG

_HIGH = jax.lax.Precision.HIGHEST


def _agg_body(y_hbm, srcs_hbm, dsts_hbm, zeros_hbm, out_hbm,
              src_v, dst_v, rows_v, acc_sh, gsem, ssem, *, groups):
    c = lax.axis_index("c")
    s = lax.axis_index("s")
    base = s * ROWS_BIG
    pltpu.sync_copy(srcs_hbm.at[c].at[s], src_v)
    pltpu.sync_copy(dsts_hbm.at[c].at[s], dst_v)

    @pl.when(s < 15)
    def _():
        pltpu.sync_copy(zeros_hbm, acc_sh.at[pl.ds(base, ROWS_BIG)])

    @pl.when(s == 15)
    def _():
        pltpu.sync_copy(zeros_hbm.at[pl.ds(0, ROWS_LAST)],
                        acc_sh.at[pl.ds(15 * ROWS_BIG, ROWS_LAST)])

    plsc.subcore_barrier()

    def step(g, carry):
        gds = []
        for b in range(NBUF):
            j = g * NBUF + b

            @pl.when(g > 0)
            def _(b=b, j=j):
                pltpu.make_async_copy(rows_v.at[b],
                                      acc_sh.at[dst_v.at[j - NBUF]],
                                      ssem.at[b]).wait()

            gds.append(pltpu.async_copy(y_hbm.at[src_v.at[j]],
                                        rows_v.at[b], gsem.at[b]))
        for b in range(NBUF):
            j = g * NBUF + b
            gds[b].wait()
            pltpu.async_copy(rows_v.at[b], acc_sh.at[dst_v.at[j]],
                             ssem.at[b], add=True)
        return carry

    lax.fori_loop(0, groups, step, 0)
    for b in range(NBUF):
        j = (groups - 1) * NBUF + b
        pltpu.make_async_copy(rows_v.at[b], acc_sh.at[dst_v.at[j]],
                              ssem.at[b]).wait()
    plsc.subcore_barrier()

    @pl.when(s < 15)
    def _():
        pltpu.sync_copy(acc_sh.at[pl.ds(base, ROWS_BIG)],
                        out_hbm.at[c].at[pl.ds(base, ROWS_BIG)])

    @pl.when(s == 15)
    def _():
        pltpu.sync_copy(acc_sh.at[pl.ds(15 * ROWS_BIG, ROWS_LAST)],
                        out_hbm.at[c].at[pl.ds(15 * ROWS_BIG, ROWS_LAST)])


def _make_agg(chunks):
    return pl.kernel(
        functools.partial(_agg_body, groups=chunks // NBUF),
        out_type=jax.ShapeDtypeStruct((NUM_CORES, N, H), jnp.float32),
        mesh=plsc.VectorSubcoreMesh(core_axis_name="c", subcore_axis_name="s"),
        scratch_types=[
            pltpu.VMEM((chunks, CHUNK), jnp.int32),
            pltpu.VMEM((chunks, CHUNK), jnp.int32),
            pltpu.VMEM((NBUF, CHUNK, H), jnp.float32),
            pltpu.VMEM_SHARED((N, H), jnp.float32),
            pltpu.SemaphoreType.DMA((NBUF,)),
            pltpu.SemaphoreType.DMA((NBUF,)),
        ],
        compiler_params=pltpu.CompilerParams(use_tc_tiling_on_sc=False),
    )


_agg = _make_agg(CHUNKS)
_agg_half = _make_agg(2 * CHUNKS)


_BLK = 2000


def _layer1_body(x_ref, p_ref, wa_ref, ba_ref, wb_ref, bb_ref, o_ref):
    agg = jnp.concatenate([p_ref[0], p_ref[1]], axis=1)
    t = x_ref[...] + agg
    t = jnp.dot(t, wa_ref[...], preferred_element_type=jnp.float32) + ba_ref[...]
    t = jnp.maximum(t, 0.0)
    t = jnp.dot(t, wb_ref[...], preferred_element_type=jnp.float32) + bb_ref[...]
    o_ref[...] = jnp.maximum(t, 0.0)


_layer1 = pl.pallas_call(
    _layer1_body,
    grid=(N // _BLK,),
    in_specs=[
        pl.BlockSpec((_BLK, F), lambda i: (i, 0)),
        pl.BlockSpec((2, _BLK, H), lambda i: (0, i, 0)),
        pl.BlockSpec((F, H), lambda i: (0, 0)),
        pl.BlockSpec((1, H), lambda i: (0, 0)),
        pl.BlockSpec((H, H), lambda i: (0, 0)),
        pl.BlockSpec((1, H), lambda i: (0, 0)),
    ],
    out_specs=pl.BlockSpec((_BLK, H), lambda i: (i, 0)),
    out_shape=jax.ShapeDtypeStruct((N, H), jnp.float32),
)


def _layer2_body(h_ref, p_ref, wa_ref, ba_ref, wb_ref, bb_ref, o_ref):
    t = h_ref[...] + p_ref[0] + p_ref[1]
    t = jnp.dot(t, wa_ref[...], preferred_element_type=jnp.float32) + ba_ref[...]
    t = jnp.maximum(t, 0.0)
    t = jnp.dot(t, wb_ref[...], preferred_element_type=jnp.float32) + bb_ref[...]
    o_ref[...] = jnp.maximum(t, 0.0)


_layer2 = pl.pallas_call(
    _layer2_body,
    grid=(N // _BLK,),
    in_specs=[
        pl.BlockSpec((_BLK, H), lambda i: (i, 0)),
        pl.BlockSpec((2, _BLK, H), lambda i: (0, i, 0)),
        pl.BlockSpec((H, H), lambda i: (0, 0)),
        pl.BlockSpec((1, H), lambda i: (0, 0)),
        pl.BlockSpec((H, H), lambda i: (0, 0)),
        pl.BlockSpec((1, H), lambda i: (0, 0)),
    ],
    out_specs=pl.BlockSpec((_BLK, H), lambda i: (i, 0)),
    out_shape=jax.ShapeDtypeStruct((N, H), jnp.float32),
)


def _layer3_pool_body(h_ref, p_ref, batch_ref, wa_ref, ba_ref, wb_ref,
                      bb_ref, wf_ref, bf_ref, o_ref, sums_ref, counts_ref):
    i = pl.program_id(0)
    t = h_ref[...] + p_ref[0] + p_ref[1]
    t = jnp.dot(t, wa_ref[...], preferred_element_type=jnp.float32) + ba_ref[...]
    t = jnp.maximum(t, 0.0)
    h3 = jnp.dot(t, wb_ref[...], preferred_element_type=jnp.float32) + bb_ref[...]
    gid = lax.broadcasted_iota(jnp.int32, (G, _BLK), 0)
    m = (gid == batch_ref[0]).astype(jnp.float32)
    blk_sums = jnp.dot(m, h3, preferred_element_type=jnp.float32,
                       precision=_HIGH)
    blk_counts = jnp.sum(m, axis=1, keepdims=True)

    @pl.when(i == 0)
    def _():
        sums_ref[...] = jnp.zeros_like(sums_ref)
        counts_ref[...] = jnp.zeros_like(counts_ref)

    sums_ref[...] += blk_sums
    counts_ref[...] += blk_counts

    @pl.when(i == pl.num_programs(0) - 1)
    def _():
        pooled = sums_ref[...] / jnp.maximum(counts_ref[...], 1.0)
        o_ref[...] = jnp.dot(pooled, wf_ref[...],
                             preferred_element_type=jnp.float32) + bf_ref[...]


_layer3_pool = pl.pallas_call(
    _layer3_pool_body,
    grid=(N // _BLK,),
    in_specs=[
        pl.BlockSpec((_BLK, H), lambda i: (i, 0)),
        pl.BlockSpec((2, _BLK, H), lambda i: (0, i, 0)),
        pl.BlockSpec((1, 1, _BLK), lambda i: (i, 0, 0)),
        pl.BlockSpec((H, H), lambda i: (0, 0)),
        pl.BlockSpec((1, H), lambda i: (0, 0)),
        pl.BlockSpec((H, H), lambda i: (0, 0)),
        pl.BlockSpec((1, H), lambda i: (0, 0)),
        pl.BlockSpec((H, 1), lambda i: (0, 0)),
        pl.BlockSpec((1, 1), lambda i: (0, 0)),
    ],
    out_specs=pl.BlockSpec((G, 1), lambda i: (0, 0)),
    out_shape=jax.ShapeDtypeStruct((G, 1), jnp.float32),
    scratch_shapes=[
        pltpu.VMEM((G, H), jnp.float32),
        pltpu.VMEM((G, 1), jnp.float32),
    ],
)


def kernel(x, edge_index, batch, W1a, b1a, W1b, b1b, W2a, b2a, W2b, b2b,
           W3a, b3a, W3b, b3b, Wf, bf):
    shp = (NUM_CORES, NUM_SUBCORES, CHUNKS, CHUNK)
    src = edge_index[0]
    src_r = src.reshape(shp)
    dst_r = edge_index[1].reshape(shp)
    zeros_h = jnp.zeros((ROWS_BIG, H), jnp.float32)
    batch2d = batch.reshape(N // _BLK, 1, _BLK)

    x2 = x.reshape(2 * N, H)
    half_shp = (NUM_SUBCORES, 2 * CHUNKS, CHUNK)
    src2 = src * 2
    src_both = jnp.stack([src2.reshape(half_shp),
                          (src2 + 1).reshape(half_shp)])
    dst_m = edge_index[1].reshape(half_shp)
    dst_both = jnp.stack([dst_m, dst_m])
    p = _agg_half(x2, src_both, dst_both, zeros_h)
    h1 = _layer1(x, p, W1a, b1a.reshape(1, H), W1b, b1b.reshape(1, H))

    p = _agg(h1, src_r, dst_r, zeros_h)
    h2 = _layer2(h1, p, W2a, b2a.reshape(1, H), W2b, b2b.reshape(1, H))

    p = _agg(h2, src_r, dst_r, zeros_h)
    return _layer3_pool(h2, p, batch2d, W3a, b3a.reshape(1, H), W3b,
                        b3b.reshape(1, H), Wf, bf.reshape(1, 1))


# speedup vs baseline: 1.4539x; 1.1210x over previous
import functools

import jax
import jax.numpy as jnp
from jax import lax
from jax.experimental import pallas as pl
from jax.experimental.pallas import tpu as pltpu
from jax.experimental.pallas import tpu_sc as plsc

N, E, F, H, G = 10000, 320000, 128, 64, 128

NUM_CORES = 2
NUM_SUBCORES = 16
NUM_WORKERS = NUM_CORES * NUM_SUBCORES
EDGES_PER_TILE = E // NUM_WORKERS
CHUNK = 80
CHUNKS = EDGES_PER_TILE // CHUNK
NBUF = 5
ROWS_BIG = 632
ROWS_LAST = N - 15 * ROWS_BIG

_HIGH = jax.lax.Precision.HIGHEST


def _agg_body(y_hbm, srcs_hbm, dsts_hbm, zeros_hbm, out_hbm,
              src_v, dst_v, rows_v, acc_sh, gsem, ssem, *, groups):
    c = lax.axis_index("c")
    s = lax.axis_index("s")
    base = s * ROWS_BIG
    pltpu.sync_copy(srcs_hbm.at[c].at[s], src_v)
    pltpu.sync_copy(dsts_hbm.at[c].at[s], dst_v)

    @pl.when(s < 15)
    def _():
        pltpu.sync_copy(zeros_hbm, acc_sh.at[pl.ds(base, ROWS_BIG)])

    @pl.when(s == 15)
    def _():
        pltpu.sync_copy(zeros_hbm.at[pl.ds(0, ROWS_LAST)],
                        acc_sh.at[pl.ds(15 * ROWS_BIG, ROWS_LAST)])

    plsc.subcore_barrier()

    def step(g, carry):
        gds = []
        for b in range(NBUF):
            j = g * NBUF + b

            @pl.when(g > 0)
            def _(b=b, j=j):
                pltpu.make_async_copy(rows_v.at[b],
                                      acc_sh.at[dst_v.at[j - NBUF]],
                                      ssem.at[b]).wait()

            gds.append(pltpu.async_copy(y_hbm.at[src_v.at[j]],
                                        rows_v.at[b], gsem.at[b]))
        for b in range(NBUF):
            j = g * NBUF + b
            gds[b].wait()
            pltpu.async_copy(rows_v.at[b], acc_sh.at[dst_v.at[j]],
                             ssem.at[b], add=True)
        return carry

    lax.fori_loop(0, groups, step, 0)
    for b in range(NBUF):
        j = (groups - 1) * NBUF + b
        pltpu.make_async_copy(rows_v.at[b], acc_sh.at[dst_v.at[j]],
                              ssem.at[b]).wait()
    plsc.subcore_barrier()

    @pl.when(s < 15)
    def _():
        pltpu.sync_copy(acc_sh.at[pl.ds(base, ROWS_BIG)],
                        out_hbm.at[c].at[pl.ds(base, ROWS_BIG)])

    @pl.when(s == 15)
    def _():
        pltpu.sync_copy(acc_sh.at[pl.ds(15 * ROWS_BIG, ROWS_LAST)],
                        out_hbm.at[c].at[pl.ds(15 * ROWS_BIG, ROWS_LAST)])


def _make_agg(chunks):
    return pl.kernel(
        functools.partial(_agg_body, groups=chunks // NBUF),
        out_type=jax.ShapeDtypeStruct((NUM_CORES, N, H), jnp.float32),
        mesh=plsc.VectorSubcoreMesh(core_axis_name="c", subcore_axis_name="s"),
        scratch_types=[
            pltpu.VMEM((chunks, CHUNK), jnp.int32),
            pltpu.VMEM((chunks, CHUNK), jnp.int32),
            pltpu.VMEM((NBUF, CHUNK, H), jnp.float32),
            pltpu.VMEM_SHARED((N, H), jnp.float32),
            pltpu.SemaphoreType.DMA((NBUF,)),
            pltpu.SemaphoreType.DMA((NBUF,)),
        ],
        compiler_params=pltpu.CompilerParams(use_tc_tiling_on_sc=False),
    )


_agg = _make_agg(CHUNKS)
_agg_half = _make_agg(2 * CHUNKS)


_BLK = 2000


def _layer1_body(x_ref, p_ref, wa_ref, ba_ref, wb_ref, bb_ref, o_ref):
    agg = jnp.concatenate([p_ref[0], p_ref[1]], axis=1)
    t = x_ref[...] + agg
    t = jnp.dot(t, wa_ref[...], preferred_element_type=jnp.float32) + ba_ref[...]
    t = jnp.maximum(t, 0.0)
    t = jnp.dot(t, wb_ref[...], preferred_element_type=jnp.float32) + bb_ref[...]
    o_ref[...] = jnp.maximum(t, 0.0)


_layer1 = pl.pallas_call(
    _layer1_body,
    grid=(N // _BLK,),
    in_specs=[
        pl.BlockSpec((_BLK, F), lambda i: (i, 0)),
        pl.BlockSpec((2, _BLK, H), lambda i: (0, i, 0)),
        pl.BlockSpec((F, H), lambda i: (0, 0)),
        pl.BlockSpec((1, H), lambda i: (0, 0)),
        pl.BlockSpec((H, H), lambda i: (0, 0)),
        pl.BlockSpec((1, H), lambda i: (0, 0)),
    ],
    out_specs=pl.BlockSpec((_BLK, H), lambda i: (i, 0)),
    out_shape=jax.ShapeDtypeStruct((N, H), jnp.float32),
)


def _layer2_body(h_ref, p_ref, wa_ref, ba_ref, wb_ref, bb_ref, o_ref):
    t = h_ref[...] + p_ref[0] + p_ref[1]
    t = jnp.dot(t, wa_ref[...], preferred_element_type=jnp.float32) + ba_ref[...]
    t = jnp.maximum(t, 0.0)
    t = jnp.dot(t, wb_ref[...], preferred_element_type=jnp.float32) + bb_ref[...]
    o_ref[...] = jnp.maximum(t, 0.0)


_layer2 = pl.pallas_call(
    _layer2_body,
    grid=(N // _BLK,),
    in_specs=[
        pl.BlockSpec((_BLK, H), lambda i: (i, 0)),
        pl.BlockSpec((2, _BLK, H), lambda i: (0, i, 0)),
        pl.BlockSpec((H, H), lambda i: (0, 0)),
        pl.BlockSpec((1, H), lambda i: (0, 0)),
        pl.BlockSpec((H, H), lambda i: (0, 0)),
        pl.BlockSpec((1, H), lambda i: (0, 0)),
    ],
    out_specs=pl.BlockSpec((_BLK, H), lambda i: (i, 0)),
    out_shape=jax.ShapeDtypeStruct((N, H), jnp.float32),
)


def _layer3_pool_body(h_ref, p_ref, batch_ref, wa_ref, ba_ref, wb_ref,
                      bb_ref, wf_ref, bf_ref, o_ref, sums_ref, counts_ref):
    i = pl.program_id(0)
    t = h_ref[...] + p_ref[0] + p_ref[1]
    t = jnp.dot(t, wa_ref[...], preferred_element_type=jnp.float32) + ba_ref[...]
    t = jnp.maximum(t, 0.0)
    h3 = jnp.dot(t, wb_ref[...], preferred_element_type=jnp.float32) + bb_ref[...]
    gid = lax.broadcasted_iota(jnp.int32, (G, _BLK), 0)
    m = (gid == batch_ref[0]).astype(jnp.float32)
    blk_sums = jnp.dot(m, h3, preferred_element_type=jnp.float32,
                       precision=_HIGH)
    blk_counts = jnp.sum(m, axis=1, keepdims=True)

    @pl.when(i == 0)
    def _():
        sums_ref[...] = jnp.zeros_like(sums_ref)
        counts_ref[...] = jnp.zeros_like(counts_ref)

    sums_ref[...] += blk_sums
    counts_ref[...] += blk_counts

    @pl.when(i == pl.num_programs(0) - 1)
    def _():
        pooled = sums_ref[...] / jnp.maximum(counts_ref[...], 1.0)
        o_ref[...] = jnp.dot(pooled, wf_ref[...],
                             preferred_element_type=jnp.float32) + bf_ref[...]


_layer3_pool = pl.pallas_call(
    _layer3_pool_body,
    grid=(N // _BLK,),
    in_specs=[
        pl.BlockSpec((_BLK, H), lambda i: (i, 0)),
        pl.BlockSpec((2, _BLK, H), lambda i: (0, i, 0)),
        pl.BlockSpec((1, 1, _BLK), lambda i: (i, 0, 0)),
        pl.BlockSpec((H, H), lambda i: (0, 0)),
        pl.BlockSpec((1, H), lambda i: (0, 0)),
        pl.BlockSpec((H, H), lambda i: (0, 0)),
        pl.BlockSpec((1, H), lambda i: (0, 0)),
        pl.BlockSpec((H, 1), lambda i: (0, 0)),
        pl.BlockSpec((1, 1), lambda i: (0, 0)),
    ],
    out_specs=pl.BlockSpec((G, 1), lambda i: (0, 0)),
    out_shape=jax.ShapeDtypeStruct((G, 1), jnp.float32),
    scratch_shapes=[
        pltpu.VMEM((G, H), jnp.float32),
        pltpu.VMEM((G, 1), jnp.float32),
    ],
)


def kernel(x, edge_index, batch, W1a, b1a, W1b, b1b, W2a, b2a, W2b, b2b,
           W3a, b3a, W3b, b3b, Wf, bf):
    shp = (NUM_CORES, NUM_SUBCORES, CHUNKS, CHUNK)
    src = edge_index[0]
    src_r = src.reshape(shp)
    dst_r = edge_index[1].reshape(shp)
    zeros_h = jnp.zeros((ROWS_BIG, H), jnp.float32)
    batch2d = batch.reshape(N // _BLK, 1, _BLK)

    x2 = x.reshape(2 * N, H)
    half_shp = (NUM_SUBCORES, 2 * CHUNKS, CHUNK)
    src2 = src * 2
    src_both = jnp.stack([src2.reshape(half_shp),
                          (src2 + 1).reshape(half_shp)])
    dst_m = edge_index[1].reshape(half_shp)
    dst_both = jnp.stack([dst_m, dst_m])
    p = _agg_half(x2, src_both, dst_both, zeros_h)
    h1 = _layer1(x, p, W1a, b1a.reshape(1, H), W1b, b1b.reshape(1, H))

    p = _agg(h1, src_r, dst_r, zeros_h)
    h2 = _layer2(h1, p, W2a, b2a.reshape(1, H), W2b, b2b.reshape(1, H))

    p = _agg(h2, src_r, dst_r, zeros_h)
    return _layer3_pool(h2, p, batch2d, W3a, b3a.reshape(1, H), W3b,
                        b3b.reshape(1, H), Wf, bf.reshape(1, 1))
